# baseline (speedup 1.0000x reference)
import functools

import jax
import jax.numpy as jnp
from jax import lax
from jax.experimental import pallas as pl
from jax.experimental.pallas import tpu as pltpu
from jax.experimental.pallas import tpu_sc as plsc

N = 320000
C = 128
NUM_SEG = 10000

NC = 2
NS = 16
NW = NC * NS
CH = 128
NCHUNKS = N // CH
BASE_CHUNKS = NCHUNKS // NW
EXTRA = NCHUNKS - BASE_CHUNKS * NW
MAXCH = BASE_CHUNKS + 1
PADCH = 80
SEG_STRIDE = 624
SEG_SPAN = 640
CNT_W = 16

_mesh = plsc.VectorSubcoreMesh(
    core_axis_name="c", subcore_axis_name="s", num_cores=NC, num_subcores=NS
)


def _worker_span(wid):
  base = BASE_CHUNKS * wid + jnp.maximum(wid - (NW - EXTRA), 0)
  nch = jnp.where(wid >= NW - EXTRA, BASE_CHUNKS + 1, BASE_CHUNKS)
  return base, nch


@functools.partial(
    pl.kernel,
    out_type=jax.ShapeDtypeStruct((NC, NUM_SEG, C), jnp.float32),
    mesh=_mesh,
    scratch_types=[
        pltpu.VMEM((CH, C), jnp.float32),
        pltpu.VMEM((CH, C), jnp.float32),
        pltpu.VMEM((PADCH, CH), jnp.int32),
        pltpu.VMEM_SHARED((NUM_SEG, C), jnp.float32),
        pltpu.SemaphoreType.DMA,
        pltpu.SemaphoreType.DMA,
    ],
)
def _seg_sum(x_hbm, idx_hbm, zsum_hbm, out_sum,
             xv0, xv1, iv, acc_s, sx0, sx1):
  cid = lax.axis_index("c")
  sid = lax.axis_index("s")
  wid = cid * NS + sid
  base, nch = _worker_span(wid)
  xv = (xv0, xv1)
  sx = (sx0, sx1)

  def x_copy(j, p):
    row0 = (base + j) * CH
    return pltpu.make_async_copy(x_hbm.at[pl.ds(row0, CH), :], xv[p], sx[p])

  seg0 = sid * SEG_STRIDE
  pltpu.sync_copy(zsum_hbm.at[pl.ds(seg0, SEG_SPAN), :],
                  acc_s.at[pl.ds(seg0, SEG_SPAN), :])
  pltpu.sync_copy(idx_hbm.at[wid], iv)
  plsc.subcore_barrier()

  x_copy(0, 0).start()

  def pair(jj, carry):
    for p in (0, 1):
      j = 2 * jj + p

      @pl.when(j < nch)
      def _():
        x_copy(j, p).wait()

        @pl.when(j + 1 < nch)
        def _():
          x_copy(j + 1, 1 - p).start()

        pltpu.sync_copy(xv[p], acc_s.at[iv.at[j]], add=True)
    return carry

  lax.fori_loop(0, (nch + 1) // 2, pair, 0)
  plsc.subcore_barrier()

  pltpu.sync_copy(acc_s.at[pl.ds(seg0, SEG_SPAN), :],
                  out_sum.at[cid, pl.ds(seg0, SEG_SPAN), :])


@functools.partial(
    pl.kernel,
    out_type=jax.ShapeDtypeStruct((NC * NUM_SEG,), jnp.float32),
    mesh=_mesh,
    scratch_types=[
        pltpu.VMEM((CH,), jnp.float32),
        pltpu.VMEM((PADCH, CH), jnp.int32),
        pltpu.VMEM((SEG_SPAN,), jnp.float32),
        pltpu.VMEM_SHARED((NUM_SEG,), jnp.float32),
    ],
)
def _seg_cnt(idx_hbm, out_cnt, ones_v, iv, stage_v, acc_c):
  cid = lax.axis_index("c")
  sid = lax.axis_index("s")
  wid = cid * NS + sid
  base, nch = _worker_span(wid)

  seg0 = sid * SEG_STRIDE

  def fz(r, carry):
    ones_v[pl.ds(r * 16, 16)] = jnp.full((16,), 0.0, jnp.float32)
    return carry

  lax.fori_loop(0, CH // 16, fz, 0)
  for k in range(SEG_SPAN // CH):
    pltpu.sync_copy(ones_v, acc_c.at[pl.ds(seg0 + k * CH, CH)])

  def f1(r, carry):
    ones_v[pl.ds(r * 16, 16)] = jnp.full((16,), 1.0, jnp.float32)
    return carry

  lax.fori_loop(0, CH // 16, f1, 0)
  pltpu.sync_copy(idx_hbm.at[wid], iv)
  plsc.subcore_barrier()

  def body(j, carry):
    pltpu.sync_copy(ones_v, acc_c.at[iv.at[j]], add=True)
    return carry

  lax.fori_loop(0, nch, body, 0)
  plsc.subcore_barrier()

  pltpu.sync_copy(acc_c.at[pl.ds(seg0, SEG_SPAN)], stage_v)
  pltpu.sync_copy(stage_v, out_cnt.at[pl.ds(cid * NUM_SEG + seg0, SEG_SPAN)])


def _mlp_body(psum_ref, pcnt_ref, w1_ref, w2_ref, s_ref):
  sums = psum_ref[0] + psum_ref[1]
  cnt = pcnt_ref[0] + pcnt_ref[1]
  pooled = sums / jnp.maximum(cnt, 1.0)
  h = lax.dot_general(pooled, w1_ref[...], (((1,), (1,)), ((), ())),
                      preferred_element_type=jnp.float32)
  h = jnp.maximum(h, 0.0)
  z = lax.dot_general(h, w2_ref[...], (((1,), (1,)), ((), ())),
                      preferred_element_type=jnp.float32)
  s_ref[...] = jax.nn.sigmoid(z)


_mlp = pl.pallas_call(
    _mlp_body,
    out_shape=jax.ShapeDtypeStruct((NUM_SEG, C), jnp.float32),
)


@functools.partial(
    pl.kernel,
    out_type=jax.ShapeDtypeStruct((N, C), jnp.float32),
    mesh=_mesh,
    scratch_types=[
        pltpu.VMEM((CH, C), jnp.float32),
        pltpu.VMEM((CH, C), jnp.float32),
        pltpu.VMEM((CH, C), jnp.float32),
        pltpu.VMEM((CH, C), jnp.float32),
        pltpu.VMEM((CH, C), jnp.float32),
        pltpu.VMEM((CH, C), jnp.float32),
        pltpu.VMEM((PADCH, CH), jnp.int32),
        pltpu.SemaphoreType.DMA,
        pltpu.SemaphoreType.DMA,
        pltpu.SemaphoreType.DMA,
        pltpu.SemaphoreType.DMA,
        pltpu.SemaphoreType.DMA,
        pltpu.SemaphoreType.DMA,
    ],
)
def _gather_mul(x_hbm, idx_hbm, s_hbm, out_hbm,
                xv0, xv1, sv0, sv1, ov0, ov1, iv,
                sx0, sx1, ss0, ss1, so0, so1):
  cid = lax.axis_index("c")
  sid = lax.axis_index("s")
  wid = cid * NS + sid
  base, nch = _worker_span(wid)
  xv = (xv0, xv1)
  sv = (sv0, sv1)
  ov = (ov0, ov1)
  sx = (sx0, sx1)
  ss = (ss0, ss1)
  so = (so0, so1)

  def x_copy(j, p):
    row0 = (base + j) * CH
    return pltpu.make_async_copy(x_hbm.at[pl.ds(row0, CH), :], xv[p], sx[p])

  def s_copy(j, p):
    return pltpu.make_async_copy(s_hbm.at[iv.at[j]], sv[p], ss[p])

  def o_copy(j, p):
    row0 = (base + j) * CH
    return pltpu.make_async_copy(ov[p], out_hbm.at[pl.ds(row0, CH), :], so[p])

  pltpu.sync_copy(idx_hbm.at[wid], iv)
  x_copy(0, 0).start()
  s_copy(0, 0).start()
  x_copy(1, 1).start()
  s_copy(1, 1).start()

  def pair(jj, carry):
    for p in (0, 1):
      j = 2 * jj + p

      @pl.when(j < nch)
      def _():
        @pl.when(j >= 2)
        def _():
          o_copy(j - 2, p).wait()

        x_copy(j, p).wait()
        s_copy(j, p).wait()

        def mul_row(r, c2):
          for cc in range(C // 16):
            sl = pl.ds(cc * 16, 16)
            ov[p][r, sl] = xv[p][r, sl] * sv[p][r, sl]
          return c2

        lax.fori_loop(0, CH, mul_row, 0)
        o_copy(j, p).start()

        @pl.when(j + 2 < nch)
        def _():
          x_copy(j + 2, p).start()
          s_copy(j + 2, p).start()
    return carry

  lax.fori_loop(0, (nch + 1) // 2, pair, 0)

  last = nch - 1
  for p in (0, 1):
    jp = jnp.where((last % 2) == p, last, last - 1)
    o_copy(jp, p).wait()


def kernel(x, indices, W1, W2):
  idx2d = indices.astype(jnp.int32).reshape(NCHUNKS, CH)
  idx_pad = jnp.concatenate(
      [idx2d, jnp.zeros((PADCH, CH), jnp.int32)], axis=0)
  planes = []
  for w in range(NW):
    b = BASE_CHUNKS * w + max(w - (NW - EXTRA), 0)
    planes.append(lax.slice(idx_pad, (b, 0), (b + PADCH, CH)))
  idx3d = jnp.stack(planes)
  zsum = jnp.zeros((NUM_SEG, C), jnp.float32)
  psum = _seg_sum(x, idx3d, zsum)
  pcnt = _seg_cnt(idx3d)
  s = _mlp(psum, pcnt.reshape(NC, NUM_SEG, 1), W1, W2)
  return _gather_mul(x, idx3d, s)


# speedup vs baseline: 1.0032x; 1.0032x over previous
import functools

import jax
import jax.numpy as jnp
from jax import lax
from jax.experimental import pallas as pl
from jax.experimental.pallas import tpu as pltpu
from jax.experimental.pallas import tpu_sc as plsc

N = 320000
C = 128
NUM_SEG = 10000

NC = 2
NS = 16
NW = NC * NS
CH = 128
NCHUNKS = N // CH
BASE_CHUNKS = NCHUNKS // NW
EXTRA = NCHUNKS - BASE_CHUNKS * NW
MAXCH = BASE_CHUNKS + 1
PADCH = 80
SEG_STRIDE = 624
SEG_SPAN = 640

_mesh = plsc.VectorSubcoreMesh(
    core_axis_name="c", subcore_axis_name="s", num_cores=NC, num_subcores=NS
)


def _worker_span(wid):
  base = BASE_CHUNKS * wid + jnp.maximum(wid - (NW - EXTRA), 0)
  nch = jnp.where(wid >= NW - EXTRA, BASE_CHUNKS + 1, BASE_CHUNKS)
  return base, nch


@functools.partial(
    pl.kernel,
    out_type=jax.ShapeDtypeStruct((NC, NUM_SEG, C), jnp.float32),
    mesh=_mesh,
    scratch_types=[
        pltpu.VMEM((CH, C), jnp.float32),
        pltpu.VMEM((CH, C), jnp.float32),
        pltpu.VMEM((PADCH, CH), jnp.int32),
        pltpu.VMEM_SHARED((NUM_SEG, C), jnp.float32),
        pltpu.SemaphoreType.DMA,
        pltpu.SemaphoreType.DMA,
    ],
)
def _seg_sum(x_hbm, idx_hbm, zsum_hbm, out_sum,
             xv0, xv1, iv, acc_s, sx0, sx1):
  cid = lax.axis_index("c")
  sid = lax.axis_index("s")
  wid = cid * NS + sid
  base, nch = _worker_span(wid)
  xv = (xv0, xv1)
  sx = (sx0, sx1)

  def x_copy(j, p):
    row0 = (base + j) * CH
    return pltpu.make_async_copy(x_hbm.at[pl.ds(row0, CH), :], xv[p], sx[p])

  seg0 = sid * SEG_STRIDE
  pltpu.sync_copy(zsum_hbm.at[pl.ds(seg0, SEG_SPAN), :],
                  acc_s.at[pl.ds(seg0, SEG_SPAN), :])
  pltpu.sync_copy(idx_hbm.at[wid], iv)
  plsc.subcore_barrier()

  x_copy(0, 0).start()

  def pair(jj, carry):
    for p in (0, 1):
      j = 2 * jj + p

      @pl.when(j < nch)
      def _():
        x_copy(j, p).wait()

        @pl.when(j + 1 < nch)
        def _():
          x_copy(j + 1, 1 - p).start()

        pltpu.sync_copy(xv[p], acc_s.at[iv.at[j]], add=True)
    return carry

  lax.fori_loop(0, (nch + 1) // 2, pair, 0)
  plsc.subcore_barrier()

  pltpu.sync_copy(acc_s.at[pl.ds(seg0, SEG_SPAN), :],
                  out_sum.at[cid, pl.ds(seg0, SEG_SPAN), :])


@functools.partial(
    pl.kernel,
    out_type=jax.ShapeDtypeStruct((NC * NUM_SEG,), jnp.float32),
    mesh=_mesh,
    scratch_types=[
        pltpu.VMEM((CH,), jnp.float32),
        pltpu.VMEM((PADCH, CH), jnp.int32),
        pltpu.VMEM((SEG_SPAN,), jnp.float32),
        pltpu.VMEM_SHARED((NUM_SEG,), jnp.float32),
    ],
)
def _seg_cnt(idx_hbm, out_cnt, ones_v, iv, stage_v, acc_c):
  cid = lax.axis_index("c")
  sid = lax.axis_index("s")
  wid = cid * NS + sid
  base, nch = _worker_span(wid)

  seg0 = sid * SEG_STRIDE

  def fz(r, carry):
    ones_v[pl.ds(r * 16, 16)] = jnp.full((16,), 0.0, jnp.float32)
    return carry

  lax.fori_loop(0, CH // 16, fz, 0)
  for k in range(SEG_SPAN // CH):
    pltpu.sync_copy(ones_v, acc_c.at[pl.ds(seg0 + k * CH, CH)])

  def f1(r, carry):
    ones_v[pl.ds(r * 16, 16)] = jnp.full((16,), 1.0, jnp.float32)
    return carry

  lax.fori_loop(0, CH // 16, f1, 0)
  pltpu.sync_copy(idx_hbm.at[wid], iv)
  plsc.subcore_barrier()

  def body(j, carry):
    pltpu.sync_copy(ones_v, acc_c.at[iv.at[j]], add=True)
    return carry

  lax.fori_loop(0, nch, body, 0)
  plsc.subcore_barrier()

  pltpu.sync_copy(acc_c.at[pl.ds(seg0, SEG_SPAN)], stage_v)
  pltpu.sync_copy(stage_v, out_cnt.at[pl.ds(cid * NUM_SEG + seg0, SEG_SPAN)])


def _mlp_body(psum_ref, pcnt_ref, w1_ref, w2_ref, s_ref):
  sums = psum_ref[0] + psum_ref[1]
  cnt = pcnt_ref[0] + pcnt_ref[1]
  pooled = sums / jnp.maximum(cnt, 1.0)
  h = lax.dot_general(pooled, w1_ref[...], (((1,), (1,)), ((), ())),
                      preferred_element_type=jnp.float32)
  h = jnp.maximum(h, 0.0)
  z = lax.dot_general(h, w2_ref[...], (((1,), (1,)), ((), ())),
                      preferred_element_type=jnp.float32)
  s_ref[...] = jax.nn.sigmoid(z)


_mlp = pl.pallas_call(
    _mlp_body,
    out_shape=jax.ShapeDtypeStruct((NUM_SEG, C), jnp.float32),
)


@functools.partial(
    pl.kernel,
    out_type=jax.ShapeDtypeStruct((N, C), jnp.float32),
    mesh=_mesh,
    scratch_types=[
        pltpu.VMEM((CH, C), jnp.float32),
        pltpu.VMEM((CH, C), jnp.float32),
        pltpu.VMEM((CH, C), jnp.float32),
        pltpu.VMEM((CH, C), jnp.float32),
        pltpu.VMEM((CH, C), jnp.float32),
        pltpu.VMEM((CH, C), jnp.float32),
        pltpu.VMEM((PADCH, CH), jnp.int32),
        pltpu.SemaphoreType.DMA,
        pltpu.SemaphoreType.DMA,
        pltpu.SemaphoreType.DMA,
        pltpu.SemaphoreType.DMA,
        pltpu.SemaphoreType.DMA,
        pltpu.SemaphoreType.DMA,
    ],
)
def _gather_mul(x_hbm, idx_hbm, s_hbm, out_hbm,
                xv0, xv1, sv0, sv1, ov0, ov1, iv,
                sx0, sx1, ss0, ss1, so0, so1):
  cid = lax.axis_index("c")
  sid = lax.axis_index("s")
  wid = cid * NS + sid
  base, nch = _worker_span(wid)
  xv = (xv0, xv1)
  sv = (sv0, sv1)
  ov = (ov0, ov1)
  sx = (sx0, sx1)
  ss = (ss0, ss1)
  so = (so0, so1)

  def x_copy(j, p):
    row0 = (base + j) * CH
    return pltpu.make_async_copy(x_hbm.at[pl.ds(row0, CH), :], xv[p], sx[p])

  def s_copy(j, p):
    return pltpu.make_async_copy(s_hbm.at[iv.at[j]], sv[p], ss[p])

  def o_copy(j, p):
    row0 = (base + j) * CH
    return pltpu.make_async_copy(ov[p], out_hbm.at[pl.ds(row0, CH), :], so[p])

  pltpu.sync_copy(idx_hbm.at[wid], iv)
  x_copy(0, 0).start()
  s_copy(0, 0).start()
  x_copy(1, 1).start()
  s_copy(1, 1).start()

  def pair(jj, carry):
    for p in (0, 1):
      j = 2 * jj + p

      @pl.when(j < nch)
      def _():
        @pl.when(j >= 2)
        def _():
          o_copy(j - 2, p).wait()

        x_copy(j, p).wait()
        s_copy(j, p).wait()

        def mul_row(r, c2):
          for cc in range(C // 16):
            sl = pl.ds(cc * 16, 16)
            ov[p][r, sl] = xv[p][r, sl] * sv[p][r, sl]
          return c2

        lax.fori_loop(0, CH, mul_row, 0)
        o_copy(j, p).start()

        @pl.when(j + 2 < nch)
        def _():
          x_copy(j + 2, p).start()
          s_copy(j + 2, p).start()
    return carry

  lax.fori_loop(0, (nch + 1) // 2, pair, 0)

  last = nch - 1
  for p in (0, 1):
    jp = jnp.where((last % 2) == p, last, last - 1)
    o_copy(jp, p).wait()


def kernel(x, indices, W1, W2):
  idx2d = indices.astype(jnp.int32).reshape(NCHUNKS, CH)
  idx_pad = jnp.concatenate(
      [idx2d, jnp.zeros((PADCH, CH), jnp.int32)], axis=0)
  planes = []
  for w in range(NW):
    b = BASE_CHUNKS * w + max(w - (NW - EXTRA), 0)
    planes.append(lax.slice(idx_pad, (b, 0), (b + PADCH, CH)))
  idx3d = jnp.stack(planes)
  zsum = jnp.zeros((NUM_SEG, C), jnp.float32)
  psum = _seg_sum(x, idx3d, zsum)
  pcnt = _seg_cnt(idx3d)
  s = _mlp(psum, pcnt.reshape(NC, NUM_SEG, 1), W1, W2)
  return _gather_mul(x, idx3d, s)
